# Initial kernel scaffold; baseline (speedup 1.0000x reference)
import jax
import jax.numpy as jnp
from jax.experimental import pallas as pl


def kernel(indices, table, W1, b1, W2, b2):
    raise NotImplementedError("write your pallas kernel here")


# speedup vs baseline: 1.1758x; 1.1758x over previous
import functools

import jax
import jax.numpy as jnp
from jax import lax
from jax.experimental import pallas as pl
from jax.experimental.pallas import tpu as pltpu
from jax.experimental.pallas import tpu_sc as plsc

VOCAB = 1000000
EMBED_DIM = 64
AUX_DIM = 128
TARGET_DIM = 64
BATCH = 16384
HIST = 50
B_TOTAL = BATCH * HIST

NC = 2
NS = 16
NW = NC * NS
B_PER_W = B_TOTAL // NW

GRP = 128
K = 4
CHUNK = GRP * K
N_CHUNKS = B_PER_W // CHUNK
IDX_ROWS_PER_W = B_PER_W // GRP


def _sc_gather_body(idx_hbm, table_hbm, out_hbm, idx_v, rows_v, sem):
    wid = lax.axis_index("s") * NC + lax.axis_index("c")
    idx_row0 = wid * IDX_ROWS_PER_W
    out_row0 = wid * B_PER_W

    def step(i, carry):
        pltpu.sync_copy(idx_hbm.at[pl.ds(idx_row0 + i * K, K)], idx_v)
        descs = [
            pltpu.async_copy(
                table_hbm.at[idx_v.at[j]],
                rows_v.at[pl.ds(j * GRP, GRP)],
                sem,
            )
            for j in range(K)
        ]
        for d in descs:
            d.wait()
        pltpu.sync_copy(rows_v, out_hbm.at[pl.ds(out_row0 + i * CHUNK, CHUNK)])
        return carry

    lax.fori_loop(0, N_CHUNKS, step, 0)


_sc_gather = pl.kernel(
    _sc_gather_body,
    out_type=jax.ShapeDtypeStruct((B_TOTAL, EMBED_DIM), jnp.float32),
    mesh=plsc.VectorSubcoreMesh(
        core_axis_name="c", subcore_axis_name="s", num_cores=NC, num_subcores=NS
    ),
    scratch_types=[
        pltpu.VMEM((K, GRP), jnp.int32),
        pltpu.VMEM((CHUNK, EMBED_DIM), jnp.float32),
        pltpu.SemaphoreType.DMA,
    ],
    compiler_params=pltpu.CompilerParams(use_tc_tiling_on_sc=False),
)


BLK = 4096


def _tc_proj_body(g_ref, w1_ref, b1_ref, w2_ref, b2_ref, o_ref):
    wct = lax.dot_general(
        w1_ref[...], w2_ref[...], (((0,), (1,)), ((), ())),
        preferred_element_type=jnp.float32,
    )
    bct = lax.dot_general(
        b1_ref[...], w2_ref[...], (((1,), (1,)), ((), ())),
        preferred_element_type=jnp.float32,
    ) + b2_ref[...]
    o_ref[...] = lax.dot_general(
        g_ref[...], wct, (((1,), (0,)), ((), ())),
        preferred_element_type=jnp.float32,
    ) + bct


_tc_proj = pl.pallas_call(
    _tc_proj_body,
    grid=(B_TOTAL // BLK,),
    in_specs=[
        pl.BlockSpec((BLK, EMBED_DIM), lambda i: (i, 0)),
        pl.BlockSpec((AUX_DIM, EMBED_DIM), lambda i: (0, 0)),
        pl.BlockSpec((1, AUX_DIM), lambda i: (0, 0)),
        pl.BlockSpec((TARGET_DIM, AUX_DIM), lambda i: (0, 0)),
        pl.BlockSpec((1, TARGET_DIM), lambda i: (0, 0)),
    ],
    out_specs=pl.BlockSpec((BLK, TARGET_DIM), lambda i: (i, 0)),
    out_shape=jax.ShapeDtypeStruct((B_TOTAL, TARGET_DIM), jnp.float32),
)


def kernel(indices, table, W1, b1, W2, b2):
    idx2d = indices.astype(jnp.int32).reshape(B_TOTAL // GRP, GRP)
    gathered = _sc_gather(idx2d, table)
    out = _tc_proj(gathered, W1, b1.reshape(1, AUX_DIM), W2, b2.reshape(1, TARGET_DIM))
    return out.reshape(BATCH, HIST, TARGET_DIM)


# speedup vs baseline: 1.5434x; 1.3126x over previous
import functools

import jax
import jax.numpy as jnp
from jax import lax
from jax.experimental import pallas as pl
from jax.experimental.pallas import tpu as pltpu
from jax.experimental.pallas import tpu_sc as plsc

VOCAB = 1000000
EMBED_DIM = 64
AUX_DIM = 128
TARGET_DIM = 64
BATCH = 16384
HIST = 50
B_TOTAL = BATCH * HIST

NC = 2
NS = 16
NW = NC * NS
B_PER_W = B_TOTAL // NW

GRP = 128
K = 4
CHUNK = GRP * K
N_CHUNKS = B_PER_W // CHUNK
IDX_ROWS_PER_W = B_PER_W // GRP


def _sc_gather_body(idx_hbm, table_hbm, out_hbm, idx_v, rows_v, sem):
    wid = lax.axis_index("s") * NC + lax.axis_index("c")
    idx_row0 = wid * IDX_ROWS_PER_W
    out_row0 = wid * B_PER_W

    def step(i, carry):
        pltpu.sync_copy(idx_hbm.at[pl.ds(idx_row0 + i * K, K)], idx_v)
        descs = [
            pltpu.async_copy(
                table_hbm.at[idx_v.at[j]],
                rows_v.at[pl.ds(j * GRP, GRP)],
                sem,
            )
            for j in range(K)
        ]
        for d in descs:
            d.wait()
        pltpu.sync_copy(rows_v, out_hbm.at[pl.ds(out_row0 + i * CHUNK, CHUNK)])
        return carry

    lax.fori_loop(0, N_CHUNKS, step, 0)


_sc_gather = pl.kernel(
    _sc_gather_body,
    out_type=jax.ShapeDtypeStruct((B_TOTAL, EMBED_DIM), jnp.float32),
    mesh=plsc.VectorSubcoreMesh(
        core_axis_name="c", subcore_axis_name="s", num_cores=NC, num_subcores=NS
    ),
    scratch_types=[
        pltpu.VMEM((K, GRP), jnp.int32),
        pltpu.VMEM((CHUNK, EMBED_DIM), jnp.float32),
        pltpu.SemaphoreType.DMA,
    ],
    compiler_params=pltpu.CompilerParams(use_tc_tiling_on_sc=False),
)


BLK = 2048
B_PAIRS = B_TOTAL // 2


def _tc_proj_body(g_ref, w1_ref, b1_ref, w2_ref, b2_ref, o_ref):
    wct = lax.dot_general(
        w1_ref[...], w2_ref[...], (((0,), (1,)), ((), ())),
        preferred_element_type=jnp.float32,
    )
    bct = lax.dot_general(
        b1_ref[...], w2_ref[...], (((1,), (1,)), ((), ())),
        preferred_element_type=jnp.float32,
    ) + b2_ref[...]
    z = jnp.zeros((EMBED_DIM, TARGET_DIM), jnp.float32)
    w_big = jnp.concatenate(
        [jnp.concatenate([wct, z], axis=1), jnp.concatenate([z, wct], axis=1)],
        axis=0,
    )
    b_big = jnp.concatenate([bct, bct], axis=1)
    o_ref[...] = lax.dot_general(
        g_ref[...], w_big, (((1,), (0,)), ((), ())),
        preferred_element_type=jnp.float32,
    ) + b_big


_tc_proj = pl.pallas_call(
    _tc_proj_body,
    grid=(B_PAIRS // BLK,),
    in_specs=[
        pl.BlockSpec((BLK, 2 * EMBED_DIM), lambda i: (i, 0)),
        pl.BlockSpec((AUX_DIM, EMBED_DIM), lambda i: (0, 0)),
        pl.BlockSpec((1, AUX_DIM), lambda i: (0, 0)),
        pl.BlockSpec((TARGET_DIM, AUX_DIM), lambda i: (0, 0)),
        pl.BlockSpec((1, TARGET_DIM), lambda i: (0, 0)),
    ],
    out_specs=pl.BlockSpec((BLK, 2 * TARGET_DIM), lambda i: (i, 0)),
    out_shape=jax.ShapeDtypeStruct((B_PAIRS, 2 * TARGET_DIM), jnp.float32),
)


def kernel(indices, table, W1, b1, W2, b2):
    idx2d = indices.astype(jnp.int32).reshape(B_TOTAL // GRP, GRP)
    gathered = _sc_gather(idx2d, table)
    g2 = gathered.reshape(B_PAIRS, 2 * EMBED_DIM)
    out = _tc_proj(g2, W1, b1.reshape(1, AUX_DIM), W2, b2.reshape(1, TARGET_DIM))
    return out.reshape(BATCH, HIST, TARGET_DIM)


# speedup vs baseline: 2.8901x; 1.8726x over previous
import jax
import jax.numpy as jnp
from jax import lax
from jax.experimental import pallas as pl
from jax.experimental.pallas import tpu as pltpu
from jax.experimental.pallas import tpu_sc as plsc

VOCAB = 1000000
EMBED_DIM = 64
AUX_DIM = 128
TARGET_DIM = 64
BATCH = 16384
HIST = 50
B_TOTAL = BATCH * HIST
HALF_B = BATCH // 2

NC = 2
NS = 16
NW = NC * NS
B_PER_W = B_TOTAL // NW

GRP = 128
K = 4
CHUNK = GRP * K
N_CHUNKS = B_PER_W // CHUNK
IDX_ROWS_PER_W = B_PER_W // GRP


def _sc_gather_body(idx_hbm, table_hbm, out_hbm, idx_v, rows_v, sem):
    wid = lax.axis_index("s") * NC + lax.axis_index("c")
    idx_row0 = wid * IDX_ROWS_PER_W
    out_row0 = wid * B_PER_W

    def step(i, carry):
        pltpu.sync_copy(idx_hbm.at[pl.ds(idx_row0 + i * K, K)], idx_v)
        descs = [
            pltpu.async_copy(
                table_hbm.at[idx_v.at[j]],
                rows_v.at[pl.ds(j * GRP, GRP)],
                sem,
            )
            for j in range(K)
        ]
        for d in descs:
            d.wait()
        pltpu.sync_copy(rows_v, out_hbm.at[pl.ds(out_row0 + i * CHUNK, CHUNK)])
        return carry

    lax.fori_loop(0, N_CHUNKS, step, 0)


_sc_gather = pl.kernel(
    _sc_gather_body,
    out_type=jax.ShapeDtypeStruct((B_TOTAL, EMBED_DIM), jnp.float32),
    mesh=plsc.VectorSubcoreMesh(
        core_axis_name="c", subcore_axis_name="s", num_cores=NC, num_subcores=NS
    ),
    scratch_types=[
        pltpu.VMEM((K, GRP), jnp.int32),
        pltpu.VMEM((CHUNK, EMBED_DIM), jnp.float32),
        pltpu.SemaphoreType.DMA,
    ],
    compiler_params=pltpu.CompilerParams(use_tc_tiling_on_sc=False),
)


PACK_V = 12800
PACK_H = PACK_V // 2
PACK_GRID = -(-VOCAB // PACK_V)
PACK_ROWS = PACK_GRID * PACK_H
VOCAB_PAD = 2 * PACK_ROWS


def _tc_pack_body(t_ref, o_ref):
    x = t_ref[...]
    t1 = jnp.transpose(x[:, :PACK_H], (1, 0))
    t2 = jnp.transpose(x[:, PACK_H:], (1, 0))
    o_ref[...] = jnp.concatenate([t1, t2], axis=1)


_tc_pack = pl.pallas_call(
    _tc_pack_body,
    grid=(PACK_GRID,),
    in_specs=[pl.BlockSpec((EMBED_DIM, PACK_V), lambda i: (0, i))],
    out_specs=pl.BlockSpec((PACK_H, 2 * EMBED_DIM), lambda i: (i, 0)),
    out_shape=jax.ShapeDtypeStruct((PACK_ROWS, 2 * EMBED_DIM), jnp.float32),
)


def _tc_proj_t_body(g_ref, w1_ref, b1_ref, w2_ref, b2d_ref, o_ref):
    wct_t = lax.dot_general(
        w2_ref[...], w1_ref[...], (((1,), (0,)), ((), ())),
        preferred_element_type=jnp.float32,
    )
    w2b = w2_ref[...] * b1_ref[...] + b2d_ref[...]
    bias_half = lax.dot_general(
        w2b, jnp.ones((HALF_B, AUX_DIM), jnp.float32), (((1,), (1,)), ((), ())),
        preferred_element_type=jnp.float32,
    )
    z = jnp.zeros((TARGET_DIM, EMBED_DIM), jnp.float32)
    w_full = jnp.concatenate(
        [jnp.concatenate([wct_t, z], axis=1), jnp.concatenate([z, wct_t], axis=1)],
        axis=0,
    )
    res2 = lax.dot_general(
        w_full, g_ref[0], (((1,), (1,)), ((), ())),
        preferred_element_type=jnp.float32,
    )
    res2 = res2 + jnp.concatenate([bias_half, bias_half], axis=0)
    res = jnp.concatenate([res2[:TARGET_DIM], res2[TARGET_DIM:]], axis=1)
    o_ref[...] = res[None]


_tc_proj_t = pl.pallas_call(
    _tc_proj_t_body,
    grid=(HIST,),
    in_specs=[
        pl.BlockSpec((1, HALF_B, 2 * EMBED_DIM), lambda i: (i, 0, 0)),
        pl.BlockSpec((AUX_DIM, EMBED_DIM), lambda i: (0, 0)),
        pl.BlockSpec((1, AUX_DIM), lambda i: (0, 0)),
        pl.BlockSpec((TARGET_DIM, AUX_DIM), lambda i: (0, 0)),
        pl.BlockSpec((TARGET_DIM, AUX_DIM), lambda i: (0, 0)),
    ],
    out_specs=pl.BlockSpec((1, TARGET_DIM, BATCH), lambda i: (i, 0, 0)),
    out_shape=jax.ShapeDtypeStruct((HIST, TARGET_DIM, BATCH), jnp.float32),
)


def kernel(indices, table, W1, b1, W2, b2):
    idx_t = indices.astype(jnp.int32).T
    rem = idx_t % PACK_V
    idx_t = 2 * (PACK_H * (idx_t // PACK_V) + rem % PACK_H) + rem // PACK_H
    idx_i = jnp.stack([idx_t[:, :HALF_B], idx_t[:, HALF_B:]], axis=-1)
    idx2d = idx_i.reshape(B_TOTAL // GRP, GRP)
    table_lin = _tc_pack(table.T).reshape(VOCAB_PAD, EMBED_DIM)
    gathered = _sc_gather(idx2d, table_lin)
    g3 = gathered.reshape(HIST, HALF_B, 2 * EMBED_DIM)
    b2d = jnp.broadcast_to((b2 / AUX_DIM).reshape(TARGET_DIM, 1),
                           (TARGET_DIM, AUX_DIM))
    out_t = _tc_proj_t(
        g3, W1, b1.reshape(1, AUX_DIM), W2, b2d
    )
    return jnp.transpose(out_t, (2, 0, 1))


# speedup vs baseline: 3.0433x; 1.0530x over previous
import jax
import jax.numpy as jnp
from jax import lax
from jax.experimental import pallas as pl
from jax.experimental.pallas import tpu as pltpu
from jax.experimental.pallas import tpu_sc as plsc

VOCAB = 1000000
EMBED_DIM = 64
AUX_DIM = 128
TARGET_DIM = 64
BATCH = 16384
HIST = 50
B_TOTAL = BATCH * HIST
HALF_B = BATCH // 2

NC = 2
NS = 16
NW = NC * NS
B_PER_W = B_TOTAL // NW

GRP = 128
K = 4
CHUNK = GRP * K
N_CHUNKS = B_PER_W // CHUNK
IDX_ROWS_PER_W = B_PER_W // GRP


def _sc_gather_body(idx_hbm, table_hbm, out_hbm,
                    idx_v0, idx_v1, rows_v0, rows_v1, sem0, sem1):
    wid = lax.axis_index("s") * NC + lax.axis_index("c")
    idx_row0 = wid * IDX_ROWS_PER_W
    out_row0 = wid * B_PER_W
    idx_v = (idx_v0, idx_v1)
    rows_v = (rows_v0, rows_v1)
    sems = (sem0, sem1)

    def fire(i, p):
        pltpu.sync_copy(idx_hbm.at[pl.ds(idx_row0 + i * K, K)], idx_v[p])
        for j in range(K):
            pltpu.async_copy(
                table_hbm.at[idx_v[p].at[j]],
                rows_v[p].at[pl.ds(j * GRP, GRP)],
                sems[p],
            )

    def drain_write(i, p):
        for j in range(K):
            pltpu.make_async_copy(
                table_hbm.at[idx_v[p].at[j]],
                rows_v[p].at[pl.ds(j * GRP, GRP)],
                sems[p],
            ).wait()
        pltpu.sync_copy(rows_v[p], out_hbm.at[pl.ds(out_row0 + i * CHUNK, CHUNK)])

    fire(0, 0)
    fire(1, 1)

    def step(j, carry):
        i = 2 * j
        drain_write(i, 0)
        fire(i + 2, 0)
        drain_write(i + 1, 1)
        fire(i + 3, 1)
        return carry

    lax.fori_loop(0, N_CHUNKS // 2 - 1, step, 0)
    i_last = N_CHUNKS - 2
    drain_write(i_last, 0)
    drain_write(i_last + 1, 1)


_sc_gather = pl.kernel(
    _sc_gather_body,
    out_type=jax.ShapeDtypeStruct((B_TOTAL, EMBED_DIM), jnp.float32),
    mesh=plsc.VectorSubcoreMesh(
        core_axis_name="c", subcore_axis_name="s", num_cores=NC, num_subcores=NS
    ),
    scratch_types=[
        pltpu.VMEM((K, GRP), jnp.int32),
        pltpu.VMEM((K, GRP), jnp.int32),
        pltpu.VMEM((CHUNK, EMBED_DIM), jnp.float32),
        pltpu.VMEM((CHUNK, EMBED_DIM), jnp.float32),
        pltpu.SemaphoreType.DMA,
        pltpu.SemaphoreType.DMA,
    ],
    compiler_params=pltpu.CompilerParams(use_tc_tiling_on_sc=False),
)


PACK_V = 12800
PACK_H = PACK_V // 2
PACK_GRID = -(-VOCAB // PACK_V)
PACK_ROWS = PACK_GRID * PACK_H
VOCAB_PAD = 2 * PACK_ROWS


def _tc_pack_body(t_ref, o_ref):
    x = t_ref[...]
    t1 = jnp.transpose(x[:, :PACK_H], (1, 0))
    t2 = jnp.transpose(x[:, PACK_H:], (1, 0))
    o_ref[...] = jnp.concatenate([t1, t2], axis=1)


_tc_pack = pl.pallas_call(
    _tc_pack_body,
    grid=(PACK_GRID,),
    in_specs=[pl.BlockSpec((EMBED_DIM, PACK_V), lambda i: (0, i))],
    out_specs=pl.BlockSpec((PACK_H, 2 * EMBED_DIM), lambda i: (i, 0)),
    out_shape=jax.ShapeDtypeStruct((PACK_ROWS, 2 * EMBED_DIM), jnp.float32),
)


def _tc_proj_t_body(g_ref, w1_ref, b1_ref, w2_ref, b2d_ref, o_ref):
    wct_t = lax.dot_general(
        w2_ref[...], w1_ref[...], (((1,), (0,)), ((), ())),
        preferred_element_type=jnp.float32,
    )
    w2b = w2_ref[...] * b1_ref[...] + b2d_ref[...]
    bias_half = lax.dot_general(
        w2b, jnp.ones((HALF_B, AUX_DIM), jnp.float32), (((1,), (1,)), ((), ())),
        preferred_element_type=jnp.float32,
    )
    z = jnp.zeros((TARGET_DIM, EMBED_DIM), jnp.float32)
    w_full = jnp.concatenate(
        [jnp.concatenate([wct_t, z], axis=1), jnp.concatenate([z, wct_t], axis=1)],
        axis=0,
    )
    res2 = lax.dot_general(
        w_full, g_ref[0], (((1,), (1,)), ((), ())),
        preferred_element_type=jnp.float32,
    )
    res2 = res2 + jnp.concatenate([bias_half, bias_half], axis=0)
    res = jnp.concatenate([res2[:TARGET_DIM], res2[TARGET_DIM:]], axis=1)
    o_ref[...] = res[None]


_tc_proj_t = pl.pallas_call(
    _tc_proj_t_body,
    grid=(HIST,),
    in_specs=[
        pl.BlockSpec((1, HALF_B, 2 * EMBED_DIM), lambda i: (i, 0, 0)),
        pl.BlockSpec((AUX_DIM, EMBED_DIM), lambda i: (0, 0)),
        pl.BlockSpec((1, AUX_DIM), lambda i: (0, 0)),
        pl.BlockSpec((TARGET_DIM, AUX_DIM), lambda i: (0, 0)),
        pl.BlockSpec((TARGET_DIM, AUX_DIM), lambda i: (0, 0)),
    ],
    out_specs=pl.BlockSpec((1, TARGET_DIM, BATCH), lambda i: (i, 0, 0)),
    out_shape=jax.ShapeDtypeStruct((HIST, TARGET_DIM, BATCH), jnp.float32),
)


def kernel(indices, table, W1, b1, W2, b2):
    idx_t = indices.astype(jnp.int32).T
    rem = idx_t % PACK_V
    idx_t = 2 * (PACK_H * (idx_t // PACK_V) + rem % PACK_H) + rem // PACK_H
    idx_i = jnp.stack([idx_t[:, :HALF_B], idx_t[:, HALF_B:]], axis=-1)
    idx2d = idx_i.reshape(B_TOTAL // GRP, GRP)
    table_lin = _tc_pack(table.T).reshape(VOCAB_PAD, EMBED_DIM)
    gathered = _sc_gather(idx2d, table_lin)
    g3 = gathered.reshape(HIST, HALF_B, 2 * EMBED_DIM)
    b2d = jnp.broadcast_to((b2 / AUX_DIM).reshape(TARGET_DIM, 1),
                           (TARGET_DIM, AUX_DIM))
    out_t = _tc_proj_t(
        g3, W1, b1.reshape(1, AUX_DIM), W2, b2d
    )
    return jnp.transpose(out_t, (2, 0, 1))
